# baseline (speedup 1.0000x reference)
import functools

import jax
import jax.numpy as jnp
import numpy as np
from jax import lax
from jax.experimental import pallas as pl
from jax.experimental.pallas import tpu as pltpu
from jax.experimental.pallas import tpu_sc as plsc

HEAD_DIM = 128
BASE = 10000
FREQ_ALLOCATION = 20
HALF = HEAD_DIM // 2
SPLIT = HALF - FREQ_ALLOCATION

SEQ = 8192
NPAIRS = 4 * SEQ
NWORK = 32
PW = NPAIRS // NWORK
CHUNK = 64
NCHUNK = PW // CHUNK
T_BLK = 2048


def _freq_phase_tables():
    inv_freq = 1.0 / (BASE ** (np.arange(0, HEAD_DIM, 2, dtype=np.float32) / HEAD_DIM))
    freq = np.zeros(HEAD_DIM, np.float32)
    phase = np.zeros(HEAD_DIM, np.float32)
    hp = np.float32(np.pi / 2)
    segs = [
        (0, np.arange(SPLIT, HALF), 0.0),
        (20, np.arange(SPLIT, HALF), hp),
        (40, np.arange(0, SPLIT, 2), 0.0),
        (62, np.arange(0, SPLIT, 2), hp),
        (84, np.arange(1, SPLIT, 2), 0.0),
        (106, np.arange(1, SPLIT, 2), hp),
    ]
    for base, js, ph in segs:
        freq[base:base + len(js)] = inv_freq[js]
        phase[base:base + len(js)] = ph
    return jnp.asarray(freq)[None], jnp.asarray(phase)[None]


def _off_table():
    off = np.zeros(256, np.int32)
    for cp in range(256):
        trig, c = divmod(cp, HEAD_DIM)
        j = c // 2
        if j >= SPLIT:
            sec, col = 0, (j - SPLIT) + 20 * trig
        elif j % 2 == 0:
            sec, col = 1, 40 + j // 2 + 22 * trig
        else:
            sec, col = 2, 84 + (j - 1) // 2 + 22 * trig
        off[cp] = sec * CHUNK * HEAD_DIM + col
    return jnp.asarray(off)


def _table_body(freq_ref, phase_ref, out_ref):
    i = pl.program_id(0)
    p = (lax.broadcasted_iota(jnp.int32, (T_BLK, HEAD_DIM), 0) + i * T_BLK)
    out_ref[...] = jnp.sin(p.astype(jnp.float32) * freq_ref[...] + phase_ref[...])


def _build_table(freqs, phases):
    return pl.pallas_call(
        _table_body,
        grid=(SEQ // T_BLK,),
        in_specs=[
            pl.BlockSpec((1, HEAD_DIM), lambda i: (0, 0)),
            pl.BlockSpec((1, HEAD_DIM), lambda i: (0, 0)),
        ],
        out_specs=pl.BlockSpec((T_BLK, HEAD_DIM), lambda i: (i, 0)),
        out_shape=jax.ShapeDtypeStruct((SEQ, HEAD_DIM), jnp.float32),
    )(freqs, phases)


def _sc_gather_kernel(table_hbm, pid_hbm, off_hbm, out_hbm,
                      pid_v, idx_t, idx_h, idx_w, gbuf, obuf, offv,
                      sem_a, sem_b, sem_oa, sem_ob):
    wid = lax.axis_index("s") * 2 + lax.axis_index("c")
    b = wid // 8
    srow = (wid % 8) * PW
    pltpu.sync_copy(off_hbm, offv)
    i16 = lax.broadcasted_iota(jnp.int32, (16,), 0)
    zero = i16 * 0
    sems = (sem_a, sem_b)
    osems = (sem_oa, sem_ob)

    def build_and_fire(chk, par, sem):
        pltpu.sync_copy(pid_hbm.at[b, pl.ds(srow + chk * CHUNK, CHUNK)], pid_v)
        for v in range(CHUNK // 16):
            row = i16 + v * 16
            idx_t[pl.ds(v * 16, 16)] = plsc.load_gather(pid_v, [row, zero])
            idx_h[pl.ds(v * 16, 16)] = plsc.load_gather(pid_v, [row, zero + 1])
            idx_w[pl.ds(v * 16, 16)] = plsc.load_gather(pid_v, [row, zero + 2])
        base = par * 3 * CHUNK
        pltpu.async_copy(table_hbm.at[idx_t], gbuf.at[pl.ds(base, CHUNK)], sem)
        pltpu.async_copy(table_hbm.at[idx_h], gbuf.at[pl.ds(base + CHUNK, CHUNK)], sem)
        pltpu.async_copy(table_hbm.at[idx_w], gbuf.at[pl.ds(base + 2 * CHUNK, CHUNK)], sem)

    def wait_gathers(par, sem):
        base = par * 3 * CHUNK
        for s in range(3):
            pltpu.make_async_copy(
                table_hbm.at[idx_t],
                gbuf.at[pl.ds(base + s * CHUNK, CHUNK)], sem).wait()

    def drain_out(sub, s0):
        pltpu.make_async_copy(
            obuf.at[sub, 0], out_hbm.at[b, pl.ds(s0, CHUNK), 0], osems[sub]).wait()
        pltpu.make_async_copy(
            obuf.at[sub, 1], out_hbm.at[b + 4, pl.ds(s0, CHUNK), 0], osems[sub]).wait()

    build_and_fire(0, 0, sem_a)

    def loop_body(i2, carry):
        for sub in range(2):
            chk = i2 * 2 + sub
            s0 = srow + chk * CHUNK
            wait_gathers(sub, sems[sub])

            @pl.when(chk + 1 < NCHUNK)
            def _():
                build_and_fire(chk + 1, 1 - sub, sems[1 - sub])

            @pl.when(i2 > 0)
            def _():
                drain_out(sub, s0)

            offs0 = tuple(offv[pl.ds(v * 16, 16)] + sub * 3 * CHUNK * HEAD_DIM
                          for v in range(16))

            @plsc.parallel_loop(0, CHUNK, carry=offs0, unroll=4)
            def pair_body(k, offs):
                for v in range(16):
                    vals = plsc.load_gather(gbuf, [zero, offs[v]])
                    obuf[sub, v // 8, k, pl.ds((v % 8) * 16, 16)] = vals
                return tuple(o + HEAD_DIM for o in offs)

            pltpu.async_copy(obuf.at[sub, 0], out_hbm.at[b, pl.ds(s0, CHUNK), 0],
                             osems[sub])
            pltpu.async_copy(obuf.at[sub, 1], out_hbm.at[b + 4, pl.ds(s0, CHUNK), 0],
                             osems[sub])
        return carry

    lax.fori_loop(0, NCHUNK // 2, loop_body, 0)
    drain_out(0, srow + (NCHUNK - 2) * CHUNK)
    drain_out(1, srow + (NCHUNK - 1) * CHUNK)


def kernel(position_ids):
    B, S, _ = position_ids.shape
    freqs, phases = _freq_phase_tables()
    table = _build_table(freqs, phases)
    offs = _off_table()

    sc = functools.partial(
        pl.kernel,
        mesh=plsc.VectorSubcoreMesh(core_axis_name="c", subcore_axis_name="s"),
        out_type=jax.ShapeDtypeStruct((2 * B, S, 1, HEAD_DIM), jnp.float32),
        scratch_types=[
            pltpu.VMEM((CHUNK, 3), jnp.int32),
            pltpu.VMEM((CHUNK,), jnp.int32),
            pltpu.VMEM((CHUNK,), jnp.int32),
            pltpu.VMEM((CHUNK,), jnp.int32),
            pltpu.VMEM((2 * 3 * CHUNK, HEAD_DIM), jnp.float32),
            pltpu.VMEM((2, 2, CHUNK, HEAD_DIM), jnp.float32),
            pltpu.VMEM((256,), jnp.int32),
            pltpu.SemaphoreType.DMA,
            pltpu.SemaphoreType.DMA,
            pltpu.SemaphoreType.DMA,
            pltpu.SemaphoreType.DMA,
        ],
        compiler_params=pltpu.CompilerParams(
            needs_layout_passes=False, use_tc_tiling_on_sc=False),
    )(_sc_gather_kernel)
    return sc(table, position_ids, offs)


# speedup vs baseline: 1.2145x; 1.2145x over previous
import functools

import jax
import jax.numpy as jnp
import numpy as np
from jax import lax
from jax.experimental import pallas as pl
from jax.experimental.pallas import tpu as pltpu
from jax.experimental.pallas import tpu_sc as plsc

HEAD_DIM = 128
BASE = 10000
FREQ_ALLOCATION = 20
HALF = HEAD_DIM // 2
SPLIT = HALF - FREQ_ALLOCATION

SEQ = 8192
NPAIRS = 4 * SEQ
NWORK = 32
PW = NPAIRS // NWORK
CHUNK = 64
NCHUNK = PW // CHUNK
T_BLK = 2048


def _freq_phase_tables():
    inv_freq = 1.0 / (BASE ** (np.arange(0, HEAD_DIM, 2, dtype=np.float32) / HEAD_DIM))
    freq = np.zeros(HEAD_DIM, np.float32)
    phase = np.zeros(HEAD_DIM, np.float32)
    hp = np.float32(np.pi / 2)
    segs = [
        (0, np.arange(SPLIT, HALF), 0.0),
        (20, np.arange(SPLIT, HALF), hp),
        (40, np.arange(0, SPLIT, 2), 0.0),
        (62, np.arange(0, SPLIT, 2), hp),
        (84, np.arange(1, SPLIT, 2), 0.0),
        (106, np.arange(1, SPLIT, 2), hp),
    ]
    for base, js, ph in segs:
        freq[base:base + len(js)] = inv_freq[js]
        phase[base:base + len(js)] = ph
    return jnp.asarray(freq)[None], jnp.asarray(phase)[None]


def _off_table():
    off = np.zeros(256, np.int32)
    for cp in range(256):
        trig, c = divmod(cp, HEAD_DIM)
        j = c // 2
        if j >= SPLIT:
            sec, col = 0, (j - SPLIT) + 20 * trig
        elif j % 2 == 0:
            sec, col = 1, 40 + j // 2 + 22 * trig
        else:
            sec, col = 2, 84 + (j - 1) // 2 + 22 * trig
        off[cp] = sec * CHUNK * HEAD_DIM + col
    return jnp.asarray(off)


def _table_body(freq_ref, phase_ref, out_ref):
    i = pl.program_id(0)
    p = (lax.broadcasted_iota(jnp.int32, (T_BLK, HEAD_DIM), 0) + i * T_BLK)
    out_ref[...] = jnp.sin(p.astype(jnp.float32) * freq_ref[...] + phase_ref[...])


def _build_table(freqs, phases):
    return pl.pallas_call(
        _table_body,
        grid=(SEQ // T_BLK,),
        in_specs=[
            pl.BlockSpec((1, HEAD_DIM), lambda i: (0, 0)),
            pl.BlockSpec((1, HEAD_DIM), lambda i: (0, 0)),
        ],
        out_specs=pl.BlockSpec((T_BLK, HEAD_DIM), lambda i: (i, 0)),
        out_shape=jax.ShapeDtypeStruct((SEQ, HEAD_DIM), jnp.float32),
    )(freqs, phases)


def _sc_gather_kernel(table_hbm, pid_hbm, off_hbm, out_hbm,
                      pid_v, idx_t, idx_h, idx_w, gbuf, obuf, offv,
                      sem_a, sem_b, sem_oa, sem_ob):
    wid = lax.axis_index("s") * 2 + lax.axis_index("c")
    b = wid // 8
    srow = (wid % 8) * PW
    pltpu.sync_copy(off_hbm, offv)
    pltpu.sync_copy(pid_hbm.at[pl.ds(wid * (3 * PW // 128), 3 * PW // 128)], pid_v)
    i16 = lax.broadcasted_iota(jnp.int32, (16,), 0)
    i3 = i16 * 3
    zero = i16 * 0
    sems = (sem_a, sem_b)
    osems = (sem_oa, sem_ob)

    def build_and_fire(chk, par, sem):
        fb = chk * (3 * CHUNK)
        for v in range(CHUNK // 16):
            g = i3 + (fb + v * 48)
            idx_t[pl.ds(v * 16, 16)] = plsc.load_gather(pid_v, [zero, g])
            idx_h[pl.ds(v * 16, 16)] = plsc.load_gather(pid_v, [zero, g + 1])
            idx_w[pl.ds(v * 16, 16)] = plsc.load_gather(pid_v, [zero, g + 2])
        base = par * 3 * CHUNK
        pltpu.async_copy(table_hbm.at[idx_t], gbuf.at[pl.ds(base, CHUNK)], sem)
        pltpu.async_copy(table_hbm.at[idx_h], gbuf.at[pl.ds(base + CHUNK, CHUNK)], sem)
        pltpu.async_copy(table_hbm.at[idx_w], gbuf.at[pl.ds(base + 2 * CHUNK, CHUNK)], sem)

    def wait_gathers(par, sem):
        base = par * 3 * CHUNK
        for s in range(3):
            pltpu.make_async_copy(
                table_hbm.at[idx_t],
                gbuf.at[pl.ds(base + s * CHUNK, CHUNK)], sem).wait()

    def drain_out(sub, s0):
        pltpu.make_async_copy(
            obuf.at[sub, 0], out_hbm.at[b, pl.ds(s0, CHUNK), 0], osems[sub]).wait()
        pltpu.make_async_copy(
            obuf.at[sub, 1], out_hbm.at[b + 4, pl.ds(s0, CHUNK), 0], osems[sub]).wait()

    build_and_fire(0, 0, sem_a)

    def loop_body(i2, carry):
        for sub in range(2):
            chk = i2 * 2 + sub
            s0 = srow + chk * CHUNK
            wait_gathers(sub, sems[sub])

            @pl.when(chk + 1 < NCHUNK)
            def _():
                build_and_fire(chk + 1, 1 - sub, sems[1 - sub])

            @pl.when(i2 > 0)
            def _():
                drain_out(sub, s0)

            offs0 = tuple(offv[pl.ds(v * 16, 16)] + sub * 3 * CHUNK * HEAD_DIM
                          for v in range(16))

            @plsc.parallel_loop(0, CHUNK, carry=offs0, unroll=4)
            def pair_body(k, offs):
                for v in range(16):
                    vals = plsc.load_gather(gbuf, [zero, offs[v]])
                    obuf[sub, v // 8, k, pl.ds((v % 8) * 16, 16)] = vals
                return tuple(o + HEAD_DIM for o in offs)

            pltpu.async_copy(obuf.at[sub, 0], out_hbm.at[b, pl.ds(s0, CHUNK), 0],
                             osems[sub])
            pltpu.async_copy(obuf.at[sub, 1], out_hbm.at[b + 4, pl.ds(s0, CHUNK), 0],
                             osems[sub])
        return carry

    lax.fori_loop(0, NCHUNK // 2, loop_body, 0)
    drain_out(0, srow + (NCHUNK - 2) * CHUNK)
    drain_out(1, srow + (NCHUNK - 1) * CHUNK)


def kernel(position_ids):
    B, S, _ = position_ids.shape
    freqs, phases = _freq_phase_tables()
    table = _build_table(freqs, phases)
    offs = _off_table()

    sc = functools.partial(
        pl.kernel,
        mesh=plsc.VectorSubcoreMesh(core_axis_name="c", subcore_axis_name="s"),
        out_type=jax.ShapeDtypeStruct((2 * B, S, 1, HEAD_DIM), jnp.float32),
        scratch_types=[
            pltpu.VMEM((3 * PW // 128, 128), jnp.int32),
            pltpu.VMEM((CHUNK,), jnp.int32),
            pltpu.VMEM((CHUNK,), jnp.int32),
            pltpu.VMEM((CHUNK,), jnp.int32),
            pltpu.VMEM((2 * 3 * CHUNK, HEAD_DIM), jnp.float32),
            pltpu.VMEM((2, 2, CHUNK, HEAD_DIM), jnp.float32),
            pltpu.VMEM((256,), jnp.int32),
            pltpu.SemaphoreType.DMA,
            pltpu.SemaphoreType.DMA,
            pltpu.SemaphoreType.DMA,
            pltpu.SemaphoreType.DMA,
        ],
        compiler_params=pltpu.CompilerParams(
            needs_layout_passes=False, use_tc_tiling_on_sc=False),
    )(_sc_gather_kernel)
    pid2 = position_ids.reshape(3 * NPAIRS // 128, 128)
    return sc(table, pid2, offs)
